# baseline (speedup 1.0000x reference)
import jax
import jax.numpy as jnp
from jax import lax
from jax.experimental import pallas as pl
from jax.experimental.pallas import tpu as pltpu
from jax.experimental.pallas import tpu_sc as plsc

B = 4
NQ = 300
QP = 304
D = 256
H = 8
L = 3
P = 4
DH = D // H
LP = L * P
C96 = H * LP
C384 = C96 * 4
SEQ = 8400
NW = 32
NPAD = B * QP * H
IPW = NPAD // NW
NCORN = 4 * LP
G = 8
NCH = IPW // G

_f32 = jnp.float32
_i32 = jnp.int32
_bf16 = jnp.bfloat16


def _vproj_body(ehs_ref, w_ref, b_ref, out_ref):
    out_ref[0] = (jnp.dot(ehs_ref[0], w_ref[...], preferred_element_type=_f32)
                  + b_ref[...]).astype(_bf16)


def _oproj_body(g_ref, w_ref, b_ref, out_ref):
    out_ref[0] = jnp.dot(g_ref[0][:NQ], w_ref[...],
                         preferred_element_type=_f32) + b_ref[...]


def _sampling_body(hs_ref, rp_ref, mx_ref, my_ref,
                   swx_ref, sbx_ref, swy_ref, sby_ref,
                   aww_ref, awb_ref, seg_ref,
                   wlf_ref, hlf_ref, offl_ref, hcol_ref,
                   e0_ref, e1_ref, e2_ref, e3_ref,
                   idx_ref, wgt_ref):
    b = pl.program_id(0)
    hs = hs_ref[0]
    doth = lambda a, m: jnp.dot(a, m, preferred_element_type=_f32,
                                precision=lax.Precision.HIGHEST)
    rp6 = rp_ref[0]
    rpx = doth(rp6, mx_ref[...])
    rpy = doth(rp6, my_ref[...])
    offx = jnp.dot(hs, swx_ref[...], preferred_element_type=_f32) + sbx_ref[...]
    offy = jnp.dot(hs, swy_ref[...], preferred_element_type=_f32) + sby_ref[...]
    aw = jnp.dot(hs, aww_ref[...], preferred_element_type=_f32) + awb_ref[...]
    e = jnp.exp(aw - jnp.max(aw))
    denom = jnp.dot(e, seg_ref[...], preferred_element_type=_f32)
    attn = e / denom

    wlf = wlf_ref[...]
    hlf = hlf_ref[...]
    px = (rpx + offx / wlf) * wlf - 0.5
    py = (rpy + offy / hlf) * hlf - 0.5
    x0 = jnp.floor(px)
    y0 = jnp.floor(py)
    fx = px - x0
    fy = py - y0
    x1 = x0 + 1.0
    y1 = y0 + 1.0

    vx0 = ((x0 >= 0.0) & (x0 <= wlf - 1.0)).astype(_f32)
    vx1 = ((x1 >= 0.0) & (x1 <= wlf - 1.0)).astype(_f32)
    vy0 = ((y0 >= 0.0) & (y0 <= hlf - 1.0)).astype(_f32)
    vy1 = ((y1 >= 0.0) & (y1 <= hlf - 1.0)).astype(_f32)

    xc0 = jnp.clip(x0, 0.0, wlf - 1.0)
    xc1 = jnp.clip(x1, 0.0, wlf - 1.0)
    yc0 = jnp.clip(y0, 0.0, hlf - 1.0)
    yc1 = jnp.clip(y1, 0.0, hlf - 1.0)

    base = jnp.float32(b * (SEQ * H)) + (offl_ref[...] * jnp.float32(H)
                                         + hcol_ref[...])
    i00 = base + (yc0 * wlf + xc0) * jnp.float32(H)
    i01 = base + (yc0 * wlf + xc1) * jnp.float32(H)
    i10 = base + (yc1 * wlf + xc0) * jnp.float32(H)
    i11 = base + (yc1 * wlf + xc1) * jnp.float32(H)

    w00 = attn * (1.0 - fx) * (1.0 - fy) * vx0 * vy0
    w01 = attn * fx * (1.0 - fy) * vx1 * vy0
    w10 = attn * (1.0 - fx) * fy * vx0 * vy1
    w11 = attn * fx * fy * vx1 * vy1

    e0, e1 = e0_ref[...], e1_ref[...]
    e2, e3 = e2_ref[...], e3_ref[...]
    idx_f = (doth(i00, e0) + doth(i01, e1) + doth(i10, e2) + doth(i11, e3))
    idx_ref[0, pl.ds(0, NQ)] = (idx_f + 0.5).astype(_i32)
    idx_ref[0, pl.ds(NQ, QP - NQ)] = jnp.zeros((QP - NQ, C384), _i32)
    wgt_ref[0, pl.ds(0, NQ)] = (doth(w00, e0) + doth(w01, e1) + doth(w10, e2)
                                + doth(w11, e3))
    wgt_ref[0, pl.ds(NQ, QP - NQ)] = jnp.zeros((QP - NQ, C384), _f32)


def _sc_gather_body(table, idxh, wh, out, idx_v, w_v, rows0, rows1, out_v,
                    sem0, sem1):
    wid = lax.axis_index("s") * 2 + lax.axis_index("c")
    base = wid * IPW
    pltpu.sync_copy(idxh.at[wid], idx_v)
    pltpu.sync_copy(wh.at[pl.ds(base, IPW)], w_v)
    rows = (rows0, rows1)
    sems = (sem0, sem1)

    def issue(c, k):
        pltpu.async_copy(table.at[idx_v.at[c]], rows[k], sems[k])

    def wait(c, k):
        pltpu.make_async_copy(table.at[idx_v.at[c]], rows[k], sems[k]).wait()

    def compute(c, k):
        for g in range(G):
            it = c * G + g
            acc = [jnp.zeros((16,), _f32) for _ in range(4)]
            wv = [w_v[it, pl.ds(16 * m, 16)] for m in range(NCORN // 16)]
            for j in range(NCORN):
                wj = wv[j // 16][j % 16]
                r = g * NCORN + j
                packed = plsc.bitcast(rows[k][r], _i32)
                lo = plsc.bitcast(packed << 16, _f32)
                hi = plsc.bitcast(packed & jnp.int32(-65536), _f32)
                acc[2 * (j % 2)] = acc[2 * (j % 2)] + wj * lo
                acc[2 * (j % 2) + 1] = acc[2 * (j % 2) + 1] + wj * hi
            out_v[it, pl.ds(0, 16)] = acc[0] + acc[2]
            out_v[it, pl.ds(16, 16)] = acc[1] + acc[3]

    issue(0, 0)

    def pair(i, carry):
        c0 = 2 * i
        issue(c0 + 1, 1)
        wait(c0, 0)
        compute(c0, 0)

        @pl.when(c0 + 2 < NCH)
        def _():
            issue(c0 + 2, 0)
        wait(c0 + 1, 1)
        compute(c0 + 1, 1)
        return carry

    lax.fori_loop(0, NCH // 2, pair, 0)
    pltpu.sync_copy(out_v, out.at[pl.ds(base, IPW)])


@jax.jit
def kernel(hidden_states, encoder_hidden_states, reference_points,
           spatial_shapes, so_w, so_b, aw_w, aw_b, vp_w, vp_b, op_w, op_b):
    ss = spatial_shapes.astype(_i32)

    col = jnp.arange(C96, dtype=_i32)
    lcol = (col % LP) // P
    hcol = col // LP
    wl_i = ss[:, 1][lcol]
    hl_i = ss[:, 0][lcol]
    sizes = ss[:, 0] * ss[:, 1]
    offs = jnp.concatenate([jnp.zeros((1,), _i32), jnp.cumsum(sizes)[:-1]])
    offl = offs[lcol].astype(_f32).reshape(1, C96)
    wl_f = wl_i.astype(_f32).reshape(1, C96)
    hl_f = hl_i.astype(_f32).reshape(1, C96)
    hcolf = hcol.astype(_f32).reshape(1, C96)

    ecols = jnp.arange(C384, dtype=_i32)
    emats = [(4 * col[:, None] + c == ecols[None, :]).astype(_f32)
             for c in range(4)]

    so_wr = so_w.reshape(D, C96, 2)
    swx, swy = so_wr[:, :, 0], so_wr[:, :, 1]
    so_br = so_b.reshape(C96, 2)
    sbx, sby = so_br[:, 0].reshape(1, C96), so_br[:, 1].reshape(1, C96)
    awb = aw_b.reshape(1, C96)
    seg = (col[:, None] // LP == col[None, :] // LP).astype(_f32)

    pos = jnp.arange(D, dtype=_i32)
    perm = (pos // DH) * DH + (pos % DH) % 2 * (DH // 2) + (pos % DH) // 2
    vp_w_p = vp_w[:, perm]
    vp_b_p = vp_b[perm]

    rr = jnp.arange(2 * L, dtype=_i32)
    mx = (rr[:, None] == 2 * lcol[None, :]).astype(_f32)
    my = (rr[:, None] == 2 * lcol[None, :] + 1).astype(_f32)
    rp6 = reference_points.reshape(B, NQ, 2 * L)

    st = 7
    seq_blk = SEQ // st
    value = pl.pallas_call(
        _vproj_body,
        grid=(B, st),
        in_specs=[
            pl.BlockSpec((1, seq_blk, D), lambda b, t: (b, t, 0)),
            pl.BlockSpec((D, D), lambda b, t: (0, 0)),
            pl.BlockSpec((1, D), lambda b, t: (0, 0)),
        ],
        out_specs=pl.BlockSpec((1, seq_blk, D), lambda b, t: (b, t, 0)),
        out_shape=jax.ShapeDtypeStruct((B, SEQ, D), _bf16),
    )(encoder_hidden_states, vp_w_p, vp_b_p.reshape(1, D))
    table = value.reshape(B * SEQ * H, DH)

    full = lambda shape: pl.BlockSpec(shape, lambda b: tuple(0 for _ in shape))
    perb = pl.BlockSpec((1, NQ, C96), lambda b: (b, 0, 0))
    perb4 = pl.BlockSpec((1, QP, C384), lambda b: (b, 0, 0))
    idx_f, wgt = pl.pallas_call(
        _sampling_body,
        grid=(B,),
        in_specs=[
            pl.BlockSpec((1, NQ, D), lambda b: (b, 0, 0)),
            pl.BlockSpec((1, NQ, 2 * L), lambda b: (b, 0, 0)),
            full((2 * L, C96)), full((2 * L, C96)),
            full((D, C96)), full((1, C96)), full((D, C96)), full((1, C96)),
            full((D, C96)), full((1, C96)), full((C96, C96)),
            full((1, C96)), full((1, C96)), full((1, C96)), full((1, C96)),
            full((C96, C384)), full((C96, C384)), full((C96, C384)),
            full((C96, C384)),
        ],
        out_specs=[perb4, perb4],
        out_shape=[jax.ShapeDtypeStruct((B, QP, C384), _i32),
                   jax.ShapeDtypeStruct((B, QP, C384), _f32)],
    )(hidden_states, rp6, mx, my, swx, sbx, swy, sby, aw_w, awb, seg,
      wl_f, hl_f, offl, hcolf, *emats)

    idx = idx_f.reshape(NW, NCH, G * NCORN)
    wgt = wgt.reshape(NPAD, NCORN)

    mesh = plsc.VectorSubcoreMesh(core_axis_name="c", subcore_axis_name="s",
                                  num_cores=2, num_subcores=16)
    gathered = pl.kernel(
        _sc_gather_body,
        out_type=jax.ShapeDtypeStruct((NPAD, DH), _f32),
        mesh=mesh,
        scratch_types=[
            pltpu.VMEM((NCH, G * NCORN), _i32),
            pltpu.VMEM((IPW, NCORN), _f32),
            pltpu.VMEM((G * NCORN, DH), _bf16),
            pltpu.VMEM((G * NCORN, DH), _bf16),
            pltpu.VMEM((IPW, DH), _f32),
            pltpu.SemaphoreType.DMA,
            pltpu.SemaphoreType.DMA,
        ],
        compiler_params=pltpu.CompilerParams(use_tc_tiling_on_sc=False,
                                             needs_layout_passes=False),
    )(table, idx, wgt)

    g = gathered.reshape(B, QP, D)
    out = pl.pallas_call(
        _oproj_body,
        grid=(B,),
        in_specs=[
            pl.BlockSpec((1, QP, D), lambda b: (b, 0, 0)),
            pl.BlockSpec((D, D), lambda b: (0, 0)),
            pl.BlockSpec((1, D), lambda b: (0, 0)),
        ],
        out_specs=pl.BlockSpec((1, NQ, D), lambda b: (b, 0, 0)),
        out_shape=jax.ShapeDtypeStruct((B, NQ, D), _f32),
    )(g, op_w, op_b.reshape(1, D))
    return out


# speedup vs baseline: 1.0193x; 1.0193x over previous
import jax
import jax.numpy as jnp
from jax import lax
from jax.experimental import pallas as pl
from jax.experimental.pallas import tpu as pltpu
from jax.experimental.pallas import tpu_sc as plsc

B = 4
NQ = 300
QP = 304
D = 256
H = 8
L = 3
P = 4
DH = D // H
LP = L * P
C96 = H * LP
C384 = C96 * 4
SEQ = 8400
NW = 32
NPAD = B * QP * H
IPW = NPAD // NW
NCORN = 4 * LP
G = 4
NCH = IPW // G

_f32 = jnp.float32
_i32 = jnp.int32
_bf16 = jnp.bfloat16


def _vproj_body(ehs_ref, w_ref, b_ref, out_ref):
    out_ref[0] = (jnp.dot(ehs_ref[0], w_ref[...], preferred_element_type=_f32)
                  + b_ref[...]).astype(_bf16)


def _oproj_body(g_ref, w_ref, b_ref, out_ref):
    out_ref[0] = jnp.dot(g_ref[0][:NQ], w_ref[...],
                         preferred_element_type=_f32) + b_ref[...]


def _sampling_body(hs_ref, rp_ref, mx_ref, my_ref,
                   swx_ref, sbx_ref, swy_ref, sby_ref,
                   aww_ref, awb_ref, seg_ref,
                   wlf_ref, hlf_ref, offl_ref, hcol_ref,
                   e0_ref, e1_ref, e2_ref, e3_ref,
                   idx_ref, wgt_ref):
    b = pl.program_id(0)
    hs = hs_ref[0]
    doth = lambda a, m: jnp.dot(a, m, preferred_element_type=_f32,
                                precision=lax.Precision.HIGHEST)
    rp6 = rp_ref[0]
    rpx = doth(rp6, mx_ref[...])
    rpy = doth(rp6, my_ref[...])
    offx = jnp.dot(hs, swx_ref[...], preferred_element_type=_f32) + sbx_ref[...]
    offy = jnp.dot(hs, swy_ref[...], preferred_element_type=_f32) + sby_ref[...]
    aw = jnp.dot(hs, aww_ref[...], preferred_element_type=_f32) + awb_ref[...]
    e = jnp.exp(aw - jnp.max(aw))
    denom = jnp.dot(e, seg_ref[...], preferred_element_type=_f32)
    attn = e / denom

    wlf = wlf_ref[...]
    hlf = hlf_ref[...]
    px = (rpx + offx / wlf) * wlf - 0.5
    py = (rpy + offy / hlf) * hlf - 0.5
    x0 = jnp.floor(px)
    y0 = jnp.floor(py)
    fx = px - x0
    fy = py - y0
    x1 = x0 + 1.0
    y1 = y0 + 1.0

    vx0 = ((x0 >= 0.0) & (x0 <= wlf - 1.0)).astype(_f32)
    vx1 = ((x1 >= 0.0) & (x1 <= wlf - 1.0)).astype(_f32)
    vy0 = ((y0 >= 0.0) & (y0 <= hlf - 1.0)).astype(_f32)
    vy1 = ((y1 >= 0.0) & (y1 <= hlf - 1.0)).astype(_f32)

    xc0 = jnp.clip(x0, 0.0, wlf - 1.0)
    xc1 = jnp.clip(x1, 0.0, wlf - 1.0)
    yc0 = jnp.clip(y0, 0.0, hlf - 1.0)
    yc1 = jnp.clip(y1, 0.0, hlf - 1.0)

    base = jnp.float32(b * (SEQ * H)) + (offl_ref[...] * jnp.float32(H)
                                         + hcol_ref[...])
    i00 = base + (yc0 * wlf + xc0) * jnp.float32(H)
    i01 = base + (yc0 * wlf + xc1) * jnp.float32(H)
    i10 = base + (yc1 * wlf + xc0) * jnp.float32(H)
    i11 = base + (yc1 * wlf + xc1) * jnp.float32(H)

    w00 = attn * (1.0 - fx) * (1.0 - fy) * vx0 * vy0
    w01 = attn * fx * (1.0 - fy) * vx1 * vy0
    w10 = attn * (1.0 - fx) * fy * vx0 * vy1
    w11 = attn * fx * fy * vx1 * vy1

    e0, e1 = e0_ref[...], e1_ref[...]
    e2, e3 = e2_ref[...], e3_ref[...]
    idx_f = (doth(i00, e0) + doth(i01, e1) + doth(i10, e2) + doth(i11, e3))
    idx_ref[0, pl.ds(0, NQ)] = (idx_f + 0.5).astype(_i32)
    idx_ref[0, pl.ds(NQ, QP - NQ)] = jnp.zeros((QP - NQ, C384), _i32)
    wgt_ref[0, pl.ds(0, NQ)] = (doth(w00, e0) + doth(w01, e1) + doth(w10, e2)
                                + doth(w11, e3))
    wgt_ref[0, pl.ds(NQ, QP - NQ)] = jnp.zeros((QP - NQ, C384), _f32)


def _sc_gather_body(table, idxh, wh, out, idx_v, w_v, rows0, rows1, out_v,
                    sem0, sem1):
    wid = lax.axis_index("s") * 2 + lax.axis_index("c")
    base = wid * IPW
    pltpu.sync_copy(idxh.at[wid], idx_v)
    pltpu.sync_copy(wh.at[pl.ds(base, IPW)], w_v)
    rows = (rows0, rows1)
    sems = (sem0, sem1)

    def issue(c, k):
        pltpu.async_copy(table.at[idx_v.at[c]], rows[k], sems[k])

    def wait(c, k):
        pltpu.make_async_copy(table.at[idx_v.at[c]], rows[k], sems[k]).wait()

    def compute(c, k):
        for g in range(G):
            it = c * G + g
            acc = [jnp.zeros((16,), _f32) for _ in range(4)]
            wv = [w_v[it, pl.ds(16 * m, 16)] for m in range(NCORN // 16)]
            for j in range(NCORN):
                wj = wv[j // 16][j % 16]
                r = g * NCORN + j
                packed = plsc.bitcast(rows[k][r], _i32)
                lo = plsc.bitcast(packed << 16, _f32)
                hi = plsc.bitcast(packed & jnp.int32(-65536), _f32)
                acc[2 * (j % 2)] = acc[2 * (j % 2)] + wj * lo
                acc[2 * (j % 2) + 1] = acc[2 * (j % 2) + 1] + wj * hi
            out_v[it, pl.ds(0, 16)] = acc[0] + acc[2]
            out_v[it, pl.ds(16, 16)] = acc[1] + acc[3]

    issue(0, 0)

    def pair(i, carry):
        c0 = 2 * i
        issue(c0 + 1, 1)
        wait(c0, 0)
        compute(c0, 0)

        @pl.when(c0 + 2 < NCH)
        def _():
            issue(c0 + 2, 0)
        wait(c0 + 1, 1)
        compute(c0 + 1, 1)
        return carry

    lax.fori_loop(0, NCH // 2, pair, 0)
    pltpu.sync_copy(out_v, out.at[pl.ds(base, IPW)])


@jax.jit
def kernel(hidden_states, encoder_hidden_states, reference_points,
           spatial_shapes, so_w, so_b, aw_w, aw_b, vp_w, vp_b, op_w, op_b):
    ss = spatial_shapes.astype(_i32)

    col = jnp.arange(C96, dtype=_i32)
    lcol = (col % LP) // P
    hcol = col // LP
    wl_i = ss[:, 1][lcol]
    hl_i = ss[:, 0][lcol]
    sizes = ss[:, 0] * ss[:, 1]
    offs = jnp.concatenate([jnp.zeros((1,), _i32), jnp.cumsum(sizes)[:-1]])
    offl = offs[lcol].astype(_f32).reshape(1, C96)
    wl_f = wl_i.astype(_f32).reshape(1, C96)
    hl_f = hl_i.astype(_f32).reshape(1, C96)
    hcolf = hcol.astype(_f32).reshape(1, C96)

    ecols = jnp.arange(C384, dtype=_i32)
    emats = [(4 * col[:, None] + c == ecols[None, :]).astype(_f32)
             for c in range(4)]

    so_wr = so_w.reshape(D, C96, 2)
    swx, swy = so_wr[:, :, 0], so_wr[:, :, 1]
    so_br = so_b.reshape(C96, 2)
    sbx, sby = so_br[:, 0].reshape(1, C96), so_br[:, 1].reshape(1, C96)
    awb = aw_b.reshape(1, C96)
    seg = (col[:, None] // LP == col[None, :] // LP).astype(_f32)

    pos = jnp.arange(D, dtype=_i32)
    perm = (pos // DH) * DH + (pos % DH) % 2 * (DH // 2) + (pos % DH) // 2
    vp_w_p = vp_w[:, perm]
    vp_b_p = vp_b[perm]

    rr = jnp.arange(2 * L, dtype=_i32)
    mx = (rr[:, None] == 2 * lcol[None, :]).astype(_f32)
    my = (rr[:, None] == 2 * lcol[None, :] + 1).astype(_f32)
    rp6 = reference_points.reshape(B, NQ, 2 * L)

    st = 7
    seq_blk = SEQ // st
    value = pl.pallas_call(
        _vproj_body,
        grid=(B, st),
        in_specs=[
            pl.BlockSpec((1, seq_blk, D), lambda b, t: (b, t, 0)),
            pl.BlockSpec((D, D), lambda b, t: (0, 0)),
            pl.BlockSpec((1, D), lambda b, t: (0, 0)),
        ],
        out_specs=pl.BlockSpec((1, seq_blk, D), lambda b, t: (b, t, 0)),
        out_shape=jax.ShapeDtypeStruct((B, SEQ, D), _bf16),
    )(encoder_hidden_states, vp_w_p, vp_b_p.reshape(1, D))
    table = value.reshape(B * SEQ * H, DH)

    full = lambda shape: pl.BlockSpec(shape, lambda b: tuple(0 for _ in shape))
    perb = pl.BlockSpec((1, NQ, C96), lambda b: (b, 0, 0))
    perb4 = pl.BlockSpec((1, QP, C384), lambda b: (b, 0, 0))
    idx_f, wgt = pl.pallas_call(
        _sampling_body,
        grid=(B,),
        in_specs=[
            pl.BlockSpec((1, NQ, D), lambda b: (b, 0, 0)),
            pl.BlockSpec((1, NQ, 2 * L), lambda b: (b, 0, 0)),
            full((2 * L, C96)), full((2 * L, C96)),
            full((D, C96)), full((1, C96)), full((D, C96)), full((1, C96)),
            full((D, C96)), full((1, C96)), full((C96, C96)),
            full((1, C96)), full((1, C96)), full((1, C96)), full((1, C96)),
            full((C96, C384)), full((C96, C384)), full((C96, C384)),
            full((C96, C384)),
        ],
        out_specs=[perb4, perb4],
        out_shape=[jax.ShapeDtypeStruct((B, QP, C384), _i32),
                   jax.ShapeDtypeStruct((B, QP, C384), _f32)],
    )(hidden_states, rp6, mx, my, swx, sbx, swy, sby, aw_w, awb, seg,
      wl_f, hl_f, offl, hcolf, *emats)

    idx = idx_f.reshape(NW, NCH, G * NCORN)
    wgt = wgt.reshape(NPAD, NCORN)

    mesh = plsc.VectorSubcoreMesh(core_axis_name="c", subcore_axis_name="s",
                                  num_cores=2, num_subcores=16)
    gathered = pl.kernel(
        _sc_gather_body,
        out_type=jax.ShapeDtypeStruct((NPAD, DH), _f32),
        mesh=mesh,
        scratch_types=[
            pltpu.VMEM((NCH, G * NCORN), _i32),
            pltpu.VMEM((IPW, NCORN), _f32),
            pltpu.VMEM((G * NCORN, DH), _bf16),
            pltpu.VMEM((G * NCORN, DH), _bf16),
            pltpu.VMEM((IPW, DH), _f32),
            pltpu.SemaphoreType.DMA,
            pltpu.SemaphoreType.DMA,
        ],
        compiler_params=pltpu.CompilerParams(use_tc_tiling_on_sc=False,
                                             needs_layout_passes=False),
    )(table, idx, wgt)

    g = gathered.reshape(B, QP, D)
    out = pl.pallas_call(
        _oproj_body,
        grid=(B,),
        in_specs=[
            pl.BlockSpec((1, QP, D), lambda b: (b, 0, 0)),
            pl.BlockSpec((D, D), lambda b: (0, 0)),
            pl.BlockSpec((1, D), lambda b: (0, 0)),
        ],
        out_specs=pl.BlockSpec((1, NQ, D), lambda b: (b, 0, 0)),
        out_shape=jax.ShapeDtypeStruct((B, NQ, D), _f32),
    )(g, op_w, op_b.reshape(1, D))
    return out


# speedup vs baseline: 1.1663x; 1.1443x over previous
import jax
import jax.numpy as jnp
from jax import lax
from jax.experimental import pallas as pl
from jax.experimental.pallas import tpu as pltpu
from jax.experimental.pallas import tpu_sc as plsc

B = 4
NQ = 300
QP = 304
D = 256
H = 8
L = 3
P = 4
DH = D // H
LP = L * P
C96 = H * LP
C384 = C96 * 4
SEQ = 8400
NW = 32
NPAD = B * QP * H
IPW = NPAD // NW
NCORN = 4 * LP
G = 4
NCH = IPW // G

_f32 = jnp.float32
_i32 = jnp.int32
_bf16 = jnp.bfloat16


def _iota2(shape, dim):
    return lax.broadcasted_iota(_i32, shape, dim)


def _vproj_body(ehs_ref, w_ref, b_ref, out_ref):
    out_ref[0] = (jnp.dot(ehs_ref[0], w_ref[...], preferred_element_type=_f32)
                  + b_ref[...]).astype(_bf16)


def _oproj_body(g_ref, w_ref, b_ref, out_ref):
    out_ref[0] = jnp.dot(g_ref[0][:NQ], w_ref[...],
                         preferred_element_type=_f32) + b_ref[...]


def _sampling_body(hs_ref, rp_ref, sow_ref, sob_ref,
                   aww_ref, awb_ref,
                   wlf_ref, hlf_ref, base_ref,
                   idx_ref, wgt_ref):
    b = pl.program_id(0)
    hs = hs_ref[0]
    dot = lambda a, m: jnp.dot(a, m, preferred_element_type=_f32)
    doth = lambda a, m: jnp.dot(a, m, preferred_element_type=_f32,
                                precision=lax.Precision.HIGHEST)

    lcol96 = (_iota2((1, C96), 1) % LP) // P
    mx = (_iota2((2 * L, C96), 0) == 2 * lcol96).astype(_f32)
    my = (_iota2((2 * L, C96), 0) == 2 * lcol96 + 1).astype(_f32)
    sx = (_iota2((2 * C96, C96), 0) == 2 * _iota2((2 * C96, C96), 1)
          ).astype(_f32)
    sy = (_iota2((2 * C96, C96), 0) == 2 * _iota2((2 * C96, C96), 1) + 1
          ).astype(_f32)
    seg = (_iota2((C96, C96), 0) // LP == _iota2((C96, C96), 1) // LP
           ).astype(_f32)
    emats = [(4 * _iota2((C96, C384), 0) + c == _iota2((C96, C384), 1)
              ).astype(_f32) for c in range(4)]

    rp6 = rp_ref[0]
    rpx = doth(rp6, mx)
    rpy = doth(rp6, my)
    offxy = dot(hs, sow_ref[...]) + sob_ref[...]
    offx = dot(offxy, sx)
    offy = dot(offxy, sy)
    aw = dot(hs, aww_ref[...]) + awb_ref[...]
    e = jnp.exp(aw - jnp.max(aw))
    attn = e / dot(e, seg)

    wlf = wlf_ref[...]
    hlf = hlf_ref[...]
    px = (rpx + offx / wlf) * wlf - 0.5
    py = (rpy + offy / hlf) * hlf - 0.5
    x0 = jnp.floor(px)
    y0 = jnp.floor(py)
    fx = px - x0
    fy = py - y0
    x1 = x0 + 1.0
    y1 = y0 + 1.0

    vx0 = ((x0 >= 0.0) & (x0 <= wlf - 1.0)).astype(_f32)
    vx1 = ((x1 >= 0.0) & (x1 <= wlf - 1.0)).astype(_f32)
    vy0 = ((y0 >= 0.0) & (y0 <= hlf - 1.0)).astype(_f32)
    vy1 = ((y1 >= 0.0) & (y1 <= hlf - 1.0)).astype(_f32)

    xc0 = jnp.clip(x0, 0.0, wlf - 1.0)
    xc1 = jnp.clip(x1, 0.0, wlf - 1.0)
    yc0 = jnp.clip(y0, 0.0, hlf - 1.0)
    yc1 = jnp.clip(y1, 0.0, hlf - 1.0)

    s00 = yc0 * wlf + xc0
    s01 = yc0 * wlf + xc1
    s10 = yc1 * wlf + xc0
    s11 = yc1 * wlf + xc1

    w00 = attn * (1.0 - fx) * (1.0 - fy) * vx0 * vy0
    w01 = attn * fx * (1.0 - fy) * vx1 * vy0
    w10 = attn * (1.0 - fx) * fy * vx0 * vy1
    w11 = attn * fx * fy * vx1 * vy1

    e0, e1, e2, e3 = emats
    sp = (dot(s00, e0) + dot(s01, e1) + dot(s10, e2) + dot(s11, e3))
    idx_f = sp * jnp.float32(H) + (base_ref[...]
                                   + jnp.float32(b * (SEQ * H)))
    idx_ref[0, pl.ds(0, NQ)] = (idx_f + 0.5).astype(_i32)
    idx_ref[0, pl.ds(NQ, QP - NQ)] = jnp.zeros((QP - NQ, C384), _i32)
    wgt_ref[0, pl.ds(0, NQ)] = (dot(w00, e0) + dot(w01, e1) + dot(w10, e2)
                                + dot(w11, e3))
    wgt_ref[0, pl.ds(NQ, QP - NQ)] = jnp.zeros((QP - NQ, C384), _f32)


def _sc_gather_body(table, idxh, wh, out, idx_v, w_v, rows0, rows1, out_v,
                    sem0, sem1):
    wid = lax.axis_index("s") * 2 + lax.axis_index("c")
    base = wid * IPW
    pltpu.sync_copy(idxh.at[wid], idx_v)
    pltpu.sync_copy(wh.at[pl.ds(base, IPW)], w_v)
    rows = (rows0, rows1)
    sems = (sem0, sem1)

    def issue(c, k):
        pltpu.async_copy(table.at[idx_v.at[c]], rows[k], sems[k])

    def wait(c, k):
        pltpu.make_async_copy(table.at[idx_v.at[c]], rows[k], sems[k]).wait()

    def compute(c, k):
        for g in range(G):
            it = c * G + g
            acc = [jnp.zeros((16,), _f32) for _ in range(4)]
            wv = [w_v[it, pl.ds(16 * m, 16)] for m in range(NCORN // 16)]
            for j in range(NCORN):
                wj = wv[j // 16][j % 16]
                r = g * NCORN + j
                packed = plsc.bitcast(rows[k][r], _i32)
                lo = plsc.bitcast(packed << 16, _f32)
                hi = plsc.bitcast(packed & jnp.int32(-65536), _f32)
                acc[2 * (j % 2)] = acc[2 * (j % 2)] + wj * lo
                acc[2 * (j % 2) + 1] = acc[2 * (j % 2) + 1] + wj * hi
            out_v[it, pl.ds(0, 16)] = acc[0] + acc[2]
            out_v[it, pl.ds(16, 16)] = acc[1] + acc[3]

    issue(0, 0)

    def pair(i, carry):
        c0 = 2 * i
        issue(c0 + 1, 1)
        wait(c0, 0)
        compute(c0, 0)

        @pl.when(c0 + 2 < NCH)
        def _():
            issue(c0 + 2, 0)
        wait(c0 + 1, 1)
        compute(c0 + 1, 1)
        return carry

    lax.fori_loop(0, NCH // 2, pair, 0)
    pltpu.sync_copy(out_v, out.at[pl.ds(base, IPW)])


@jax.jit
def kernel(hidden_states, encoder_hidden_states, reference_points,
           spatial_shapes, so_w, so_b, aw_w, aw_b, vp_w, vp_b, op_w, op_b):
    ss = spatial_shapes.astype(_i32)

    col = jnp.arange(C96, dtype=_i32)
    lcol = (col % LP) // P
    hcol = col // LP
    wl_i = ss[:, 1][lcol]
    hl_i = ss[:, 0][lcol]
    sizes = ss[:, 0] * ss[:, 1]
    offs = jnp.concatenate([jnp.zeros((1,), _i32), jnp.cumsum(sizes)[:-1]])
    wl_f = wl_i.astype(_f32).reshape(1, C96)
    hl_f = hl_i.astype(_f32).reshape(1, C96)
    base96 = offs[lcol] * H + hcol
    base384 = jnp.repeat(base96, 4).astype(_f32).reshape(1, C384)

    awb = aw_b.reshape(1, C96)
    sob = so_b.reshape(1, 2 * C96)

    pos = jnp.arange(D, dtype=_i32)
    perm = (pos // DH) * DH + (pos % DH) % 2 * (DH // 2) + (pos % DH) // 2
    vp_w_p = vp_w[:, perm]
    vp_b_p = vp_b[perm]

    rp6 = reference_points.reshape(B, NQ, 2 * L)

    st = 3
    seq_blk = SEQ // st
    value = pl.pallas_call(
        _vproj_body,
        grid=(B, st),
        in_specs=[
            pl.BlockSpec((1, seq_blk, D), lambda b, t: (b, t, 0)),
            pl.BlockSpec((D, D), lambda b, t: (0, 0)),
            pl.BlockSpec((1, D), lambda b, t: (0, 0)),
        ],
        out_specs=pl.BlockSpec((1, seq_blk, D), lambda b, t: (b, t, 0)),
        out_shape=jax.ShapeDtypeStruct((B, SEQ, D), _bf16),
    )(encoder_hidden_states, vp_w_p, vp_b_p.reshape(1, D))
    table = value.reshape(B * SEQ * H, DH)

    full = lambda shape: pl.BlockSpec(shape, lambda b: tuple(0 for _ in shape))
    perb4 = pl.BlockSpec((1, QP, C384), lambda b: (b, 0, 0))
    idx_f, wgt = pl.pallas_call(
        _sampling_body,
        grid=(B,),
        in_specs=[
            pl.BlockSpec((1, NQ, D), lambda b: (b, 0, 0)),
            pl.BlockSpec((1, NQ, 2 * L), lambda b: (b, 0, 0)),
            full((D, 2 * C96)), full((1, 2 * C96)),
            full((D, C96)), full((1, C96)),
            full((1, C96)), full((1, C96)), full((1, C384)),
        ],
        out_specs=[perb4, perb4],
        out_shape=[jax.ShapeDtypeStruct((B, QP, C384), _i32),
                   jax.ShapeDtypeStruct((B, QP, C384), _f32)],
    )(hidden_states, rp6, so_w, sob, aw_w, awb, wl_f, hl_f, base384)

    idx = idx_f.reshape(NW, NCH, G * NCORN)
    wgt = wgt.reshape(NPAD, NCORN)

    mesh = plsc.VectorSubcoreMesh(core_axis_name="c", subcore_axis_name="s",
                                  num_cores=2, num_subcores=16)
    gathered = pl.kernel(
        _sc_gather_body,
        out_type=jax.ShapeDtypeStruct((NPAD, DH), _f32),
        mesh=mesh,
        scratch_types=[
            pltpu.VMEM((NCH, G * NCORN), _i32),
            pltpu.VMEM((IPW, NCORN), _f32),
            pltpu.VMEM((G * NCORN, DH), _bf16),
            pltpu.VMEM((G * NCORN, DH), _bf16),
            pltpu.VMEM((IPW, DH), _f32),
            pltpu.SemaphoreType.DMA,
            pltpu.SemaphoreType.DMA,
        ],
        compiler_params=pltpu.CompilerParams(use_tc_tiling_on_sc=False,
                                             needs_layout_passes=False),
    )(table, idx, wgt)

    g = gathered.reshape(B, QP, D)
    out = pl.pallas_call(
        _oproj_body,
        grid=(B,),
        in_specs=[
            pl.BlockSpec((1, QP, D), lambda b: (b, 0, 0)),
            pl.BlockSpec((D, D), lambda b: (0, 0)),
            pl.BlockSpec((1, D), lambda b: (0, 0)),
        ],
        out_specs=pl.BlockSpec((1, NQ, D), lambda b: (b, 0, 0)),
        out_shape=jax.ShapeDtypeStruct((B, NQ, D), _f32),
    )(g, op_w, op_b.reshape(1, D))
    return out
